# Initial kernel scaffold; baseline (speedup 1.0000x reference)
import jax
import jax.numpy as jnp
from jax.experimental import pallas as pl


def kernel(x, edge_index, W1, b1, ln_g, ln_b, Wmu, bmu, Wlv, blv, Wc1, bc1, Wc2, bc2, Wm1, bm1, Wm2, bm2, tau):
    raise NotImplementedError("write your pallas kernel here")


# speedup vs baseline: 1.0119x; 1.0119x over previous
import jax
import jax.numpy as jnp
from jax.experimental import pallas as pl

N_NODES = 10000
N_EDGES = 320000
NEG_RATIO = 5
BETA = 1.0


def _gcn_plain(x, src, dst, W, b):
    h = x @ W
    out = jnp.zeros((x.shape[0], W.shape[1]), x.dtype).at[dst].add(h[src])
    return out + b


def _gcn_norm(x, src, dst, w, W, b):
    n = x.shape[0]
    loop = jnp.arange(n, dtype=src.dtype)
    s = jnp.concatenate([src, loop])
    d = jnp.concatenate([dst, loop])
    wf = jnp.concatenate([w, jnp.ones((n,), x.dtype)])
    deg = jnp.zeros((n,), x.dtype).at[d].add(wf)
    dinv = jnp.where(deg > 0.0, deg ** -0.5, 0.0)
    norm = dinv[s] * dinv[d] * wf
    h = x @ W
    out = jnp.zeros((n, W.shape[1]), x.dtype).at[d].add(norm[:, None] * h[s])
    return out + b


def _layer_norm(x, g, b, eps=1e-5):
    mu = jnp.mean(x, axis=-1, keepdims=True)
    var = jnp.mean((x - mu) ** 2, axis=-1, keepdims=True)
    return (x - mu) / jnp.sqrt(var + eps) * g + b


def _sample_negatives(key, n_neg, m):
    span = jnp.uint32(N_NODES)
    mult = jnp.uint32(2 ** 16) % span
    mult = (mult * mult) % span
    off = n_neg.astype(jnp.uint32)
    k1, k2 = jax.random.split(key)

    def rows(k):
        flat = jax.random.bits(k, (2 * m,), jnp.uint32)
        r0 = flat[:m]
        r1 = jax.lax.dynamic_slice(flat, (off.astype(jnp.int32),), (m,))
        return r0, r1

    hi0, hi1 = rows(k1)
    lo0, lo1 = rows(k2)

    def to_node(hi, lo):
        v = ((hi % span) * mult + (lo % span)) % span
        return v.astype(jnp.int32)

    return to_node(hi0, lo0), to_node(hi1, lo1)


def _combine_kernel(a_ref, out_ref):
    out_ref[...] = a_ref[...]


def kernel(x, edge_index, W1, b1, ln_g, ln_b, Wmu, bmu, Wlv, blv, Wc1, bc1, Wc2, bc2, Wm1, bm1, Wm2, bm2, tau):
    src, dst = edge_index[0], edge_index[1]
    h1 = jax.nn.relu(_layer_norm(_gcn_plain(x, src, dst, W1, b1), ln_g, ln_b))
    mu = _gcn_plain(h1, src, dst, Wmu, bmu)
    logvar = _gcn_plain(h1, src, dst, Wlv, blv)
    eps = jax.random.normal(jax.random.key(42), mu.shape, mu.dtype)
    z = mu + eps * jnp.exp(0.5 * logvar)
    pos_mask = src < dst
    wpos = pos_mask.astype(x.dtype)
    n_pos = jnp.sum(pos_mask).astype(jnp.int32)
    n_neg = jnp.int32(NEG_RATIO) * n_pos
    m = NEG_RATIO * N_EDGES
    neg0, neg1 = _sample_negatives(jax.random.key(7), n_neg, m)
    h = jax.nn.relu(_gcn_norm(z, src, dst, wpos, Wc1, bc1))
    h = jax.nn.relu(_gcn_norm(h, src, dst, wpos, Wc2, bc2))

    def edge_logits(u, v):
        hu = h[u]; hv = h[v]
        phi = jnp.concatenate([hu, hv, jnp.abs(hu - hv), hu * hv], axis=-1)
        logit = (jax.nn.relu(phi @ Wm1 + bm1) @ Wm2 + bm2)[:, 0]
        return logit / jnp.clip(tau, 1e-4)

    logits_pos = edge_logits(src, dst)
    logits_neg = edge_logits(neg0, neg1)
    wneg = (jnp.arange(m) < n_neg).astype(jnp.float32)
    n_pos_f = n_pos.astype(jnp.float32)
    n_neg_f = n_neg.astype(jnp.float32)
    pos_weight = n_neg_f / n_pos_f
    ls = jax.nn.log_sigmoid
    total = jnp.sum(wpos * (-(pos_weight * ls(logits_pos)))) + jnp.sum(wneg * (-ls(-logits_neg)))
    recon = total / (n_pos_f + n_neg_f)
    kl = -0.5 * jnp.mean(1.0 + logvar - mu ** 2 - jnp.exp(logvar))
    loss = recon + BETA * kl

    packed = jnp.stack([recon + BETA * kl, recon, kl]).reshape(1, 3)
    out = pl.pallas_call(
        _combine_kernel,
        out_shape=jax.ShapeDtypeStruct((1, 3), jnp.float32),
    )(packed)
    return (out[0, 0], out[0, 1], out[0, 2])


# speedup vs baseline: 3.5660x; 3.5241x over previous
import functools

import jax
import jax.numpy as jnp
from jax import lax
from jax.experimental import pallas as pl
from jax.experimental.pallas import tpu as pltpu
from jax.experimental.pallas import tpu_sc as plsc

N_NODES = 10000
N_EDGES = 320000
NEG_RATIO = 5
BETA = 1.0
M_NEG = NEG_RATIO * N_EDGES

NC, NS = 2, 16
NW = NC * NS
NROWS = 10112
RPT = NROWS // NS
CHUNK = 80
NB = 6400


def _sc_scatter(E, W):
    epw = E // NW
    nch = epw // CHUNK
    mesh = plsc.VectorSubcoreMesh(core_axis_name="c", subcore_axis_name="s")

    @functools.partial(
        pl.kernel,
        out_type=jax.ShapeDtypeStruct((NC, NROWS, W), jnp.float32),
        mesh=mesh,
        compiler_params=pltpu.CompilerParams(use_tc_tiling_on_sc=False),
        scratch_types=[
            pltpu.VMEM((CHUNK,), jnp.int32),
            pltpu.VMEM((CHUNK,), jnp.int32),
            pltpu.VMEM((CHUNK, W), jnp.float32),
            pltpu.VMEM((RPT, W), jnp.float32),
            pltpu.VMEM_SHARED((NROWS, W), jnp.float32),
            pltpu.SemaphoreType.DMA,
        ],
    )
    def k(table, src, dst, zrows, out, src_v, dst_v, rows_v, buf_v, acc, sem):
        cid = lax.axis_index("c")
        sid = lax.axis_index("s")
        wid = sid * NC + cid
        pltpu.sync_copy(zrows.at[pl.ds(sid * RPT, RPT)], buf_v)
        pltpu.sync_copy(buf_v, acc.at[pl.ds(sid * RPT, RPT)])
        plsc.subcore_barrier()
        base = wid * epw

        def body(g, carry):
            off = base + g * CHUNK
            pltpu.sync_copy(src.at[pl.ds(off, CHUNK)], src_v)
            pltpu.sync_copy(dst.at[pl.ds(off, CHUNK)], dst_v)
            pltpu.async_copy(table.at[src_v], rows_v, sem).wait()
            pltpu.sync_copy(rows_v, acc.at[dst_v], add=True)
            return carry

        lax.fori_loop(0, nch, body, 0)
        plsc.subcore_barrier()
        pltpu.sync_copy(acc.at[pl.ds(sid * RPT, RPT)], buf_v)
        pltpu.sync_copy(buf_v, out.at[cid, pl.ds(sid * RPT, RPT)])

    return k


MULT = ((2 ** 16) % N_NODES) ** 2 % N_NODES


def _sc_gather():
    ppw = N_EDGES // NW
    npw = M_NEG // NW
    pch = ppw // CHUNK
    nchn = npw // CHUNK
    mesh = plsc.VectorSubcoreMesh(core_axis_name="c", subcore_axis_name="s")

    @functools.partial(
        pl.kernel,
        out_type=(
            jax.ShapeDtypeStruct((N_EDGES, 32), jnp.float32),
            jax.ShapeDtypeStruct((N_EDGES, 32), jnp.float32),
            jax.ShapeDtypeStruct((M_NEG, 32), jnp.float32),
            jax.ShapeDtypeStruct((M_NEG, 32), jnp.float32),
        ),
        mesh=mesh,
        compiler_params=pltpu.CompilerParams(use_tc_tiling_on_sc=False),
        scratch_types=[
            pltpu.VMEM((CHUNK,), jnp.int32),
            pltpu.VMEM((CHUNK,), jnp.int32),
            pltpu.VMEM((CHUNK,), jnp.int32),
            pltpu.VMEM((CHUNK,), jnp.int32),
            pltpu.VMEM((CHUNK,), jnp.int32),
            pltpu.VMEM((CHUNK,), jnp.int32),
            pltpu.VMEM((CHUNK, 32), jnp.float32),
            pltpu.VMEM((CHUNK, 32), jnp.float32),
            pltpu.SemaphoreType.DMA,
        ],
    )
    def k(h, src, dst, hi0, hi1, lo0, lo1, hup, hvp, hun, hvn,
          u_v, v_v, a_v, b_v, c_v, d_v, ur_v, vr_v, sem):
        cid = lax.axis_index("c")
        sid = lax.axis_index("s")
        wid = sid * NC + cid

        def emit(uo, vo, off):
            cu = pltpu.async_copy(h.at[u_v], ur_v, sem)
            cv = pltpu.async_copy(h.at[v_v], vr_v, sem)
            cu.wait()
            cv.wait()
            pltpu.sync_copy(ur_v, uo.at[pl.ds(off, CHUNK)])
            pltpu.sync_copy(vr_v, vo.at[pl.ds(off, CHUNK)])

        pbase = wid * ppw

        def pbody(g, carry):
            off = pbase + g * CHUNK
            pltpu.sync_copy(src.at[pl.ds(off, CHUNK)], u_v)
            pltpu.sync_copy(dst.at[pl.ds(off, CHUNK)], v_v)
            emit(hup, hvp, off)
            return carry

        lax.fori_loop(0, pch, pbody, 0)

        nbase = wid * npw

        def nbody(g, carry):
            off = nbase + g * CHUNK
            pltpu.sync_copy(hi0.at[pl.ds(off, CHUNK)], a_v)
            pltpu.sync_copy(lo0.at[pl.ds(off, CHUNK)], b_v)
            pltpu.sync_copy(hi1.at[pl.ds(off, CHUNK)], c_v)
            pltpu.sync_copy(lo1.at[pl.ds(off, CHUNK)], d_v)
            for j in range(CHUNK // 16):
                sl = pl.ds(j * 16, 16)

                def umod(w):
                    hi16 = lax.shift_right_logical(w, 16)
                    lo16 = lax.bitwise_and(w, 0xFFFF)
                    return (hi16 * ((1 << 16) % N_NODES) + lo16) % N_NODES

                u_v[sl] = (umod(a_v[sl]) * MULT + umod(b_v[sl])) % N_NODES
                v_v[sl] = (umod(c_v[sl]) * MULT + umod(d_v[sl])) % N_NODES
            emit(hun, hvn, off)
            return carry

        lax.fori_loop(0, nchn, nbody, 0)

    return k


def _tc_prep_kernel(x_ref, w1_ref, ei_ref, xw_ref, dpos_ref, npos_ref):
    xw_ref[...] = jnp.dot(x_ref[...], w1_ref[...],
                          preferred_element_type=jnp.float32)
    s = ei_ref[0]
    d = ei_ref[1]
    mask = s < d
    dpos_ref[...] = jnp.where(mask, d, N_NODES)
    npos_ref[...] = jnp.sum(mask.astype(jnp.int32)).reshape(1, 1)


def _tc_enc2_kernel(p_ref, b1_ref, g_ref, bb_ref, wcat_ref, hm_ref):
    acc = (p_ref[0] + p_ref[1])[:N_NODES] + b1_ref[...]
    mu = jnp.mean(acc, axis=-1, keepdims=True)
    var = jnp.mean((acc - mu) ** 2, axis=-1, keepdims=True)
    h1 = (acc - mu) / jnp.sqrt(var + 1e-5) * g_ref[...] + bb_ref[...]
    h1 = jnp.maximum(h1, 0.0)
    hm_ref[...] = jnp.dot(h1, wcat_ref[...], preferred_element_type=jnp.float32)


def _tc_enc3_kernel(q_ref, bmu_ref, blv_ref, eps_ref, degp_ref, wc1_ref,
                    zs_ref, dinv_ref, kl_ref):
    qs = (q_ref[0] + q_ref[1])[:N_NODES]
    mu = qs[:, :32] + bmu_ref[...]
    lv = qs[:, 32:] + blv_ref[...]
    kl_ref[...] = (-0.5 * jnp.mean(1.0 + lv - mu * mu - jnp.exp(lv))).reshape(1, 1)
    z = mu + eps_ref[...] * jnp.exp(0.5 * lv)
    deg = (degp_ref[0, :, 0:1] + degp_ref[1, :, 0:1])[:N_NODES] + 1.0
    dinv = deg ** -0.5
    dinv_ref[...] = dinv
    zs_ref[...] = dinv * jnp.dot(z, wc1_ref[...],
                                 preferred_element_type=jnp.float32)


def _tc_dec_kernel(rp_ref, zs_ref, dinv_ref, b_ref, w_ref, out_ref):
    acc = (rp_ref[0] + rp_ref[1])[:N_NODES]
    dinv = dinv_ref[...]
    hd = jnp.maximum(dinv * (acc + zs_ref[...]) + b_ref[...], 0.0)
    out_ref[...] = dinv * jnp.dot(hd, w_ref[...],
                                  preferred_element_type=jnp.float32)


def _tc_dec_last_kernel(rp_ref, zs_ref, dinv_ref, b_ref, out_ref):
    acc = (rp_ref[0] + rp_ref[1])[:N_NODES]
    out_ref[...] = jnp.maximum(
        dinv_ref[...] * (acc + zs_ref[...]) + b_ref[...], 0.0)


def _edge_logit(hu, hv, wm1_ref, bm1_ref, wm2_ref, bm2_ref, tau_ref):
    phi = jnp.concatenate([hu, hv, jnp.abs(hu - hv), hu * hv], axis=1)
    act = jnp.maximum(
        jnp.dot(phi, wm1_ref[...], preferred_element_type=jnp.float32)
        + bm1_ref[...], 0.0)
    raw = jnp.dot(act, wm2_ref[...],
                  preferred_element_type=jnp.float32) + bm2_ref[0, 0]
    return raw / jnp.maximum(tau_ref[0, 0], 1e-4)


def _log_sigmoid(x):
    return jnp.minimum(x, 0.0) - jnp.log1p(jnp.exp(-jnp.abs(x)))


def _tc_score_pos_kernel(hu_ref, hv_ref, s_ref, d_ref, wm1_ref, bm1_ref,
                         wm2_ref, bm2_ref, tau_ref, out_ref):
    i = pl.program_id(0)
    logit = _edge_logit(hu_ref[...], hv_ref[...], wm1_ref, bm1_ref,
                        wm2_ref, bm2_ref, tau_ref)
    mask = s_ref[...] < d_ref[...]
    part = jnp.sum(jnp.where(mask, _log_sigmoid(logit), 0.0))

    @pl.when(i == 0)
    def _():
        out_ref[...] = jnp.zeros((1, 1), jnp.float32)

    out_ref[...] += part.reshape(1, 1)


def _tc_score_neg_kernel(hu_ref, hv_ref, nn_ref, wm1_ref, bm1_ref,
                         wm2_ref, bm2_ref, tau_ref, out_ref):
    i = pl.program_id(0)
    logit = _edge_logit(hu_ref[...], hv_ref[...], wm1_ref, bm1_ref,
                        wm2_ref, bm2_ref, tau_ref)
    j = lax.broadcasted_iota(jnp.int32, (NB, 1), 0) + i * NB
    mask = j < nn_ref[0, 0]
    part = jnp.sum(jnp.where(mask, _log_sigmoid(-logit), 0.0))

    @pl.when(i == 0)
    def _():
        out_ref[...] = jnp.zeros((1, 1), jnp.float32)

    out_ref[...] += part.reshape(1, 1)


def _full(shape):
    return pl.BlockSpec(shape, lambda i: tuple(0 for _ in shape))


def kernel(x, edge_index, W1, b1, ln_g, ln_b, Wmu, bmu, Wlv, blv,
           Wc1, bc1, Wc2, bc2, Wm1, bm1, Wm2, bm2, tau):
    f32 = jnp.float32
    src = edge_index[0]
    dst = edge_index[1]
    ei3 = edge_index.reshape(2, N_EDGES // 128, 128)

    xw, dpos2d, npos = pl.pallas_call(
        _tc_prep_kernel,
        out_shape=(
            jax.ShapeDtypeStruct((N_NODES, 64), f32),
            jax.ShapeDtypeStruct((N_EDGES // 128, 128), jnp.int32),
            jax.ShapeDtypeStruct((1, 1), jnp.int32),
        ),
    )(x, W1, ei3)
    dpos = dpos2d.reshape(N_EDGES)

    zrows64 = jnp.zeros((NROWS, 64), f32)
    zrows32 = jnp.zeros((NROWS, 32), f32)
    zrows16 = jnp.zeros((NROWS, 16), f32)
    ones16 = jnp.ones((N_NODES, 16), f32)

    p = _sc_scatter(N_EDGES, 64)(xw, src, dst, zrows64)

    wcat = jnp.concatenate([Wmu, Wlv], axis=1)
    hm = pl.pallas_call(
        _tc_enc2_kernel,
        out_shape=jax.ShapeDtypeStruct((N_NODES, 64), f32),
    )(p, b1.reshape(1, 64), ln_g.reshape(1, 64), ln_b.reshape(1, 64), wcat)

    q = _sc_scatter(N_EDGES, 64)(hm, src, dst, zrows64)
    degp = _sc_scatter(N_EDGES, 16)(ones16, src, dpos, zrows16)

    eps = jax.random.normal(jax.random.key(42), (N_NODES, 32), f32)
    zs1, dinv, kl = pl.pallas_call(
        _tc_enc3_kernel,
        out_shape=(
            jax.ShapeDtypeStruct((N_NODES, 32), f32),
            jax.ShapeDtypeStruct((N_NODES, 1), f32),
            jax.ShapeDtypeStruct((1, 1), f32),
        ),
    )(q, bmu.reshape(1, 32), blv.reshape(1, 32), eps, degp, Wc1)

    r1 = _sc_scatter(N_EDGES, 32)(zs1, src, dpos, zrows32)
    zs2 = pl.pallas_call(
        _tc_dec_kernel,
        out_shape=jax.ShapeDtypeStruct((N_NODES, 32), f32),
    )(r1, zs1, dinv, bc1.reshape(1, 32), Wc2)

    r2 = _sc_scatter(N_EDGES, 32)(zs2, src, dpos, zrows32)
    h = pl.pallas_call(
        _tc_dec_last_kernel,
        out_shape=jax.ShapeDtypeStruct((N_NODES, 32), f32),
    )(r2, zs2, dinv, bc2.reshape(1, 32))

    n_pos = npos[0, 0]
    n_neg = jnp.int32(NEG_RATIO) * n_pos
    k1, k2 = jax.random.split(jax.random.key(7))
    if jax.config.jax_threefry_partitionable:
        flat1 = jax.random.bits(k1, (2 * M_NEG,), jnp.uint32)
        flat2 = jax.random.bits(k2, (2 * M_NEG,), jnp.uint32)
        hi0u = flat1[:M_NEG]
        hi1u = lax.dynamic_slice(flat1, (n_neg,), (M_NEG,))
        lo0u = flat2[:M_NEG]
        lo1u = lax.dynamic_slice(flat2, (n_neg,), (M_NEG,))
    else:
        idx = jnp.arange(M_NEG, dtype=jnp.uint32)
        offu = n_neg.astype(jnp.uint32)
        kd1 = jax.random.key_data(k1)
        kd2 = jax.random.key_data(k2)
        hi0u, hi1u = _threefry2x32(kd1[0], kd1[1], idx, idx + offu)
        lo0u, lo1u = _threefry2x32(kd2[0], kd2[1], idx, idx + offu)
    hi0 = lax.bitcast_convert_type(hi0u, jnp.int32)
    hi1 = lax.bitcast_convert_type(hi1u, jnp.int32)
    lo0 = lax.bitcast_convert_type(lo0u, jnp.int32)
    lo1 = lax.bitcast_convert_type(lo1u, jnp.int32)

    hup, hvp, hun, hvn = _sc_gather()(h, src, dst, hi0, hi1, lo0, lo1)

    wblocks = [_full((128, 64)), _full((1, 64)), _full((64, 1)),
               _full((1, 1)), _full((1, 1))]
    wargs = (Wm1, bm1.reshape(1, 64), Wm2, bm2.reshape(1, 1),
             tau.reshape(1, 1))

    sum_pos = pl.pallas_call(
        _tc_score_pos_kernel,
        grid=(N_EDGES // NB,),
        in_specs=[
            pl.BlockSpec((NB, 32), lambda i: (i, 0)),
            pl.BlockSpec((NB, 32), lambda i: (i, 0)),
            pl.BlockSpec((NB, 1), lambda i: (i, 0)),
            pl.BlockSpec((NB, 1), lambda i: (i, 0)),
        ] + wblocks,
        out_specs=pl.BlockSpec((1, 1), lambda i: (0, 0)),
        out_shape=jax.ShapeDtypeStruct((1, 1), f32),
    )(hup, hvp, src.reshape(N_EDGES, 1), dst.reshape(N_EDGES, 1), *wargs)

    sum_neg = pl.pallas_call(
        _tc_score_neg_kernel,
        grid=(M_NEG // NB,),
        in_specs=[
            pl.BlockSpec((NB, 32), lambda i: (i, 0)),
            pl.BlockSpec((NB, 32), lambda i: (i, 0)),
            _full((1, 1)),
        ] + wblocks,
        out_specs=pl.BlockSpec((1, 1), lambda i: (0, 0)),
        out_shape=jax.ShapeDtypeStruct((1, 1), f32),
    )(hun, hvn, n_neg.reshape(1, 1), *wargs)

    n_pos_f = n_pos.astype(f32)
    n_neg_f = n_neg.astype(f32)
    pos_weight = n_neg_f / n_pos_f
    total = -pos_weight * sum_pos[0, 0] - sum_neg[0, 0]
    recon = total / (n_pos_f + n_neg_f)
    klv = kl[0, 0]
    loss = recon + BETA * klv
    return (loss, recon, klv)


def _threefry2x32(k0, k1, x0, x1):
    rots = ((13, 15, 26, 6), (17, 29, 16, 24))

    def rotl(v, d):
        return (v << jnp.uint32(d)) | (v >> jnp.uint32(32 - d))

    ks = (k0, k1, k0 ^ k1 ^ jnp.uint32(0x1BD11BDA))
    x0 = x0 + ks[0]
    x1 = x1 + ks[1]
    for i in range(5):
        for r in rots[i % 2]:
            x0 = x0 + x1
            x1 = rotl(x1, r)
            x1 = x1 ^ x0
        x0 = x0 + ks[(i + 1) % 3]
        x1 = x1 + ks[(i + 2) % 3] + jnp.uint32(i + 1)
    return x0, x1


# speedup vs baseline: 5.4600x; 1.5311x over previous
import functools

import jax
import jax.numpy as jnp
from jax import lax
from jax.experimental import pallas as pl
from jax.experimental.pallas import tpu as pltpu
from jax.experimental.pallas import tpu_sc as plsc

N_NODES = 10000
N_EDGES = 320000
NEG_RATIO = 5
BETA = 1.0
M_NEG = NEG_RATIO * N_EDGES

NC, NS = 2, 16
NW = NC * NS
NROWS = 10112
RPT = NROWS // NS
CHUNK = 80
NB = 6400


def _sc_scatter(E, W):
    epw = E // NW
    nch = epw // CHUNK
    npair = nch // 2
    mesh = plsc.VectorSubcoreMesh(core_axis_name="c", subcore_axis_name="s")

    @functools.partial(
        pl.kernel,
        out_type=jax.ShapeDtypeStruct((NC, NROWS, W), jnp.float32),
        mesh=mesh,
        compiler_params=pltpu.CompilerParams(use_tc_tiling_on_sc=False),
        scratch_types=[
            pltpu.VMEM((epw,), jnp.int32),
            pltpu.VMEM((CHUNK,), jnp.int32),
            pltpu.VMEM((CHUNK,), jnp.int32),
            pltpu.VMEM((CHUNK, W), jnp.float32),
            pltpu.VMEM((CHUNK, W), jnp.float32),
            pltpu.VMEM((RPT, W), jnp.float32),
            pltpu.VMEM_SHARED((NROWS, W), jnp.float32),
            pltpu.SemaphoreType.DMA,
        ],
    )
    def k(table, src, dst, zrows, out, spi_v, d0_v, d1_v, r0_v, r1_v,
          buf_v, acc, sem):
        cid = lax.axis_index("c")
        sid = lax.axis_index("s")
        wid = sid * NC + cid
        base = wid * epw
        pltpu.sync_copy(src.at[pl.ds(base, epw)], spi_v)
        pltpu.sync_copy(zrows.at[pl.ds(sid * RPT, RPT)], buf_v)
        pltpu.sync_copy(buf_v, acc.at[pl.ds(sid * RPT, RPT)])
        plsc.subcore_barrier()

        def pair(p, carry):
            l0 = p * (2 * CHUNK)
            c0 = pltpu.async_copy(table.at[spi_v.at[pl.ds(l0, CHUNK)]],
                                  r0_v, sem)
            c1 = pltpu.async_copy(table.at[spi_v.at[pl.ds(l0 + CHUNK, CHUNK)]],
                                  r1_v, sem)
            pltpu.sync_copy(dst.at[pl.ds(base + l0, CHUNK)], d0_v)
            pltpu.sync_copy(dst.at[pl.ds(base + l0 + CHUNK, CHUNK)], d1_v)
            c0.wait()
            pltpu.sync_copy(r0_v, acc.at[d0_v], add=True)
            c1.wait()
            pltpu.sync_copy(r1_v, acc.at[d1_v], add=True)
            return carry

        lax.fori_loop(0, npair, pair, 0)
        if nch % 2 == 1:
            l0 = npair * 2 * CHUNK
            c0 = pltpu.async_copy(table.at[spi_v.at[pl.ds(l0, CHUNK)]],
                                  r0_v, sem)
            pltpu.sync_copy(dst.at[pl.ds(base + l0, CHUNK)], d0_v)
            c0.wait()
            pltpu.sync_copy(r0_v, acc.at[d0_v], add=True)
        plsc.subcore_barrier()
        pltpu.sync_copy(acc.at[pl.ds(sid * RPT, RPT)], buf_v)
        pltpu.sync_copy(buf_v, out.at[cid, pl.ds(sid * RPT, RPT)])

    return k


MULT = ((2 ** 16) % N_NODES) ** 2 % N_NODES
SB = 10000


def _sc_gather():
    ppw = N_EDGES // NW
    npw = M_NEG // NW
    pch = ppw // CHUNK
    nsb = npw // SB
    mesh = plsc.VectorSubcoreMesh(core_axis_name="c", subcore_axis_name="s")

    @functools.partial(
        pl.kernel,
        out_type=(
            jax.ShapeDtypeStruct((N_EDGES, 32), jnp.float32),
            jax.ShapeDtypeStruct((N_EDGES, 32), jnp.float32),
            jax.ShapeDtypeStruct((M_NEG, 32), jnp.float32),
            jax.ShapeDtypeStruct((M_NEG, 32), jnp.float32),
        ),
        mesh=mesh,
        compiler_params=pltpu.CompilerParams(
            use_tc_tiling_on_sc=False, needs_layout_passes=False),
        scratch_types=[
            pltpu.VMEM((ppw,), jnp.int32),
            pltpu.VMEM((ppw,), jnp.int32),
            pltpu.VMEM((SB + 2 * CHUNK,), jnp.int32),
            pltpu.VMEM((SB,), jnp.int32),
            pltpu.VMEM((SB + 2 * CHUNK,), jnp.int32),
            pltpu.VMEM((SB,), jnp.int32),
            pltpu.VMEM((16,), jnp.int32),
            pltpu.VMEM((CHUNK, 32), jnp.float32),
            pltpu.VMEM((CHUNK, 32), jnp.float32),
            pltpu.VMEM((CHUNK, 32), jnp.float32),
            pltpu.VMEM((CHUNK, 32), jnp.float32),
            pltpu.SemaphoreType.DMA,
            pltpu.SemaphoreType.DMA,
        ],
    )
    def k(h, src, dst, hi0, hi1, lo0, lo1, nneg, hup, hvp, hun, hvn,
          spi_v, dpi_v, a_v, b_v, c_v, d_v, nn_v,
          ur0, vr0, ur1, vr1, gsem, wsem):
        cid = lax.axis_index("c")
        sid = lax.axis_index("s")
        wid = sid * NC + cid

        def pipe(iu, iv, ou, ov, goff, nch):
            npair = nch // 2

            def pair(p, carry):
                l0 = p * (2 * CHUNK)
                o0 = goff + l0
                c0u = pltpu.async_copy(h.at[iu.at[pl.ds(l0, CHUNK)]], ur0, gsem)
                c0v = pltpu.async_copy(h.at[iv.at[pl.ds(l0, CHUNK)]], vr0, gsem)
                c1u = pltpu.async_copy(h.at[iu.at[pl.ds(l0 + CHUNK, CHUNK)]],
                                       ur1, gsem)
                c1v = pltpu.async_copy(h.at[iv.at[pl.ds(l0 + CHUNK, CHUNK)]],
                                       vr1, gsem)
                c0u.wait()
                c0v.wait()
                w0 = pltpu.async_copy(ur0, ou.at[pl.ds(o0, CHUNK)], wsem)
                w1 = pltpu.async_copy(vr0, ov.at[pl.ds(o0, CHUNK)], wsem)
                c1u.wait()
                c1v.wait()
                w2 = pltpu.async_copy(ur1, ou.at[pl.ds(o0 + CHUNK, CHUNK)], wsem)
                w3 = pltpu.async_copy(vr1, ov.at[pl.ds(o0 + CHUNK, CHUNK)], wsem)
                w0.wait()
                w1.wait()
                w2.wait()
                w3.wait()
                return carry

            lax.fori_loop(0, npair, pair, 0)

            @pl.when(nch % 2 == 1)
            def _():
                l0 = npair * (2 * CHUNK)
                o0 = goff + l0
                cu = pltpu.async_copy(h.at[iu.at[pl.ds(l0, CHUNK)]], ur0, gsem)
                cv = pltpu.async_copy(h.at[iv.at[pl.ds(l0, CHUNK)]], vr0, gsem)
                cu.wait()
                cv.wait()
                pltpu.sync_copy(ur0, ou.at[pl.ds(o0, CHUNK)])
                pltpu.sync_copy(vr0, ov.at[pl.ds(o0, CHUNK)])

        pbase = wid * ppw
        pltpu.sync_copy(src.at[pl.ds(pbase, ppw)], spi_v)
        pltpu.sync_copy(dst.at[pl.ds(pbase, ppw)], dpi_v)
        pipe(spi_v, dpi_v, hup, hvp, pbase, pch)

        pltpu.sync_copy(nneg, nn_v)
        n_need = jnp.max(nn_v[...])
        zero16 = jnp.zeros((16,), jnp.int32)
        for t in range(2 * CHUNK // 16):
            a_v[pl.ds(SB + t * 16, 16)] = zero16
            c_v[pl.ds(SB + t * 16, 16)] = zero16
        nbase = wid * npw
        for sb in range(nsb):
            boff = nbase + sb * SB
            needed = jnp.clip(n_need - boff, 0, SB)
            nch = (needed + (CHUNK - 1)) // CHUNK

            @pl.when(nch > 0)
            def _(sb=sb, boff=boff, nch=nch):
                pltpu.sync_copy(hi0.at[pl.ds(boff, SB)],
                                a_v.at[pl.ds(0, SB)])
                pltpu.sync_copy(lo0.at[pl.ds(boff, SB)], b_v)
                pltpu.sync_copy(hi1.at[pl.ds(boff, SB)],
                                c_v.at[pl.ds(0, SB)])
                pltpu.sync_copy(lo1.at[pl.ds(boff, SB)], d_v)

                def conv(g, carry):
                    sl = pl.ds(g * 16, 16)

                    def umod(w):
                        hi16 = lax.shift_right_logical(w, 16)
                        lo16 = lax.bitwise_and(w, 0xFFFF)
                        return (hi16 * ((1 << 16) % N_NODES) + lo16) % N_NODES

                    a_v[sl] = (umod(a_v[sl]) * MULT + umod(b_v[sl])) % N_NODES
                    c_v[sl] = (umod(c_v[sl]) * MULT + umod(d_v[sl])) % N_NODES
                    return carry

                lax.fori_loop(0, SB // 16, conv, 0)
                pipe(a_v, c_v, hun, hvn, boff, nch)

    return k


def _tc_prep_kernel(x_ref, w1_ref, ei_ref, xw_ref, dpos_ref, npos_ref):
    xw_ref[...] = jnp.dot(x_ref[...], w1_ref[...],
                          preferred_element_type=jnp.float32)
    s = ei_ref[0]
    d = ei_ref[1]
    mask = s < d
    dpos_ref[...] = jnp.where(mask, d, N_NODES)
    npos_ref[...] = jnp.sum(mask.astype(jnp.int32)).reshape(1, 1)


def _tc_enc2_kernel(p_ref, b1_ref, g_ref, bb_ref, wcat_ref, hm_ref):
    acc = (p_ref[0] + p_ref[1])[:N_NODES] + b1_ref[...]
    mu = jnp.mean(acc, axis=-1, keepdims=True)
    var = jnp.mean((acc - mu) ** 2, axis=-1, keepdims=True)
    h1 = (acc - mu) / jnp.sqrt(var + 1e-5) * g_ref[...] + bb_ref[...]
    h1 = jnp.maximum(h1, 0.0)
    hm_ref[...] = jnp.dot(h1, wcat_ref[...], preferred_element_type=jnp.float32)


def _tc_enc3_kernel(q_ref, bmu_ref, blv_ref, eps_ref, degp_ref, wc1_ref,
                    zs_ref, dinv_ref, kl_ref):
    qs = (q_ref[0] + q_ref[1])[:N_NODES]
    mu = qs[:, :32] + bmu_ref[...]
    lv = qs[:, 32:] + blv_ref[...]
    kl_ref[...] = (-0.5 * jnp.mean(1.0 + lv - mu * mu - jnp.exp(lv))).reshape(1, 1)
    z = mu + eps_ref[...] * jnp.exp(0.5 * lv)
    deg = (degp_ref[0, :, 0:1] + degp_ref[1, :, 0:1])[:N_NODES] + 1.0
    dinv = deg ** -0.5
    dinv_ref[...] = dinv
    zs_ref[...] = dinv * jnp.dot(z, wc1_ref[...],
                                 preferred_element_type=jnp.float32)


def _tc_dec_kernel(rp_ref, zs_ref, dinv_ref, b_ref, w_ref, out_ref):
    acc = (rp_ref[0] + rp_ref[1])[:N_NODES]
    dinv = dinv_ref[...]
    hd = jnp.maximum(dinv * (acc + zs_ref[...]) + b_ref[...], 0.0)
    out_ref[...] = dinv * jnp.dot(hd, w_ref[...],
                                  preferred_element_type=jnp.float32)


def _tc_dec_last_kernel(rp_ref, zs_ref, dinv_ref, b_ref, out_ref):
    acc = (rp_ref[0] + rp_ref[1])[:N_NODES]
    out_ref[...] = jnp.maximum(
        dinv_ref[...] * (acc + zs_ref[...]) + b_ref[...], 0.0)


def _edge_logit(hu, hv, wm1_ref, bm1_ref, wm2_ref, bm2_ref, tau_ref):
    phi = jnp.concatenate([hu, hv, jnp.abs(hu - hv), hu * hv], axis=1)
    act = jnp.maximum(
        jnp.dot(phi, wm1_ref[...], preferred_element_type=jnp.float32)
        + bm1_ref[...], 0.0)
    raw = jnp.dot(act, wm2_ref[...],
                  preferred_element_type=jnp.float32) + bm2_ref[0, 0]
    return raw / jnp.maximum(tau_ref[0, 0], 1e-4)


def _log_sigmoid(x):
    return jnp.minimum(x, 0.0) - jnp.log1p(jnp.exp(-jnp.abs(x)))


def _tc_score_pos_kernel(hu_ref, hv_ref, s_ref, d_ref, wm1_ref, bm1_ref,
                         wm2_ref, bm2_ref, tau_ref, out_ref):
    i = pl.program_id(0)
    logit = _edge_logit(hu_ref[...], hv_ref[...], wm1_ref, bm1_ref,
                        wm2_ref, bm2_ref, tau_ref)
    mask = s_ref[...] < d_ref[...]
    part = jnp.sum(jnp.where(mask, _log_sigmoid(logit), 0.0))

    @pl.when(i == 0)
    def _():
        out_ref[...] = jnp.zeros((1, 1), jnp.float32)

    out_ref[...] += part.reshape(1, 1)


def _tc_score_neg_kernel(hu_ref, hv_ref, nn_ref, wm1_ref, bm1_ref,
                         wm2_ref, bm2_ref, tau_ref, out_ref):
    i = pl.program_id(0)
    logit = _edge_logit(hu_ref[...], hv_ref[...], wm1_ref, bm1_ref,
                        wm2_ref, bm2_ref, tau_ref)
    j = lax.broadcasted_iota(jnp.int32, (NB, 1), 0) + i * NB
    mask = j < nn_ref[0, 0]
    part = jnp.sum(jnp.where(mask, _log_sigmoid(-logit), 0.0))

    @pl.when(i == 0)
    def _():
        out_ref[...] = jnp.zeros((1, 1), jnp.float32)

    out_ref[...] += part.reshape(1, 1)


def _full(shape):
    return pl.BlockSpec(shape, lambda i: tuple(0 for _ in shape))


def kernel(x, edge_index, W1, b1, ln_g, ln_b, Wmu, bmu, Wlv, blv,
           Wc1, bc1, Wc2, bc2, Wm1, bm1, Wm2, bm2, tau):
    f32 = jnp.float32
    src = edge_index[0]
    dst = edge_index[1]
    ei3 = edge_index.reshape(2, N_EDGES // 128, 128)

    xw, dpos2d, npos = pl.pallas_call(
        _tc_prep_kernel,
        out_shape=(
            jax.ShapeDtypeStruct((N_NODES, 64), f32),
            jax.ShapeDtypeStruct((N_EDGES // 128, 128), jnp.int32),
            jax.ShapeDtypeStruct((1, 1), jnp.int32),
        ),
    )(x, W1, ei3)
    dpos = dpos2d.reshape(N_EDGES)

    zrows64 = jnp.zeros((NROWS, 64), f32)
    zrows32 = jnp.zeros((NROWS, 32), f32)
    zrows16 = jnp.zeros((NROWS, 16), f32)
    ones16 = jnp.ones((N_NODES, 16), f32)

    p = _sc_scatter(N_EDGES, 64)(xw, src, dst, zrows64)

    wcat = jnp.concatenate([Wmu, Wlv], axis=1)
    hm = pl.pallas_call(
        _tc_enc2_kernel,
        out_shape=jax.ShapeDtypeStruct((N_NODES, 64), f32),
    )(p, b1.reshape(1, 64), ln_g.reshape(1, 64), ln_b.reshape(1, 64), wcat)

    q = _sc_scatter(N_EDGES, 64)(hm, src, dst, zrows64)
    degp = _sc_scatter(N_EDGES, 16)(ones16, src, dpos, zrows16)

    eps = jax.random.normal(jax.random.key(42), (N_NODES, 32), f32)
    zs1, dinv, kl = pl.pallas_call(
        _tc_enc3_kernel,
        out_shape=(
            jax.ShapeDtypeStruct((N_NODES, 32), f32),
            jax.ShapeDtypeStruct((N_NODES, 1), f32),
            jax.ShapeDtypeStruct((1, 1), f32),
        ),
    )(q, bmu.reshape(1, 32), blv.reshape(1, 32), eps, degp, Wc1)

    r1 = _sc_scatter(N_EDGES, 32)(zs1, src, dpos, zrows32)
    zs2 = pl.pallas_call(
        _tc_dec_kernel,
        out_shape=jax.ShapeDtypeStruct((N_NODES, 32), f32),
    )(r1, zs1, dinv, bc1.reshape(1, 32), Wc2)

    r2 = _sc_scatter(N_EDGES, 32)(zs2, src, dpos, zrows32)
    h = pl.pallas_call(
        _tc_dec_last_kernel,
        out_shape=jax.ShapeDtypeStruct((N_NODES, 32), f32),
    )(r2, zs2, dinv, bc2.reshape(1, 32))

    n_pos = npos[0, 0]
    n_neg = jnp.int32(NEG_RATIO) * n_pos
    k1, k2 = jax.random.split(jax.random.key(7))
    if jax.config.jax_threefry_partitionable:
        flat1 = jax.random.bits(k1, (2 * M_NEG,), jnp.uint32)
        flat2 = jax.random.bits(k2, (2 * M_NEG,), jnp.uint32)
        hi0u = flat1[:M_NEG]
        hi1u = lax.dynamic_slice(flat1, (n_neg,), (M_NEG,))
        lo0u = flat2[:M_NEG]
        lo1u = lax.dynamic_slice(flat2, (n_neg,), (M_NEG,))
    else:
        idx = jnp.arange(M_NEG, dtype=jnp.uint32)
        offu = n_neg.astype(jnp.uint32)
        kd1 = jax.random.key_data(k1)
        kd2 = jax.random.key_data(k2)
        hi0u, hi1u = _threefry2x32(kd1[0], kd1[1], idx, idx + offu)
        lo0u, lo1u = _threefry2x32(kd2[0], kd2[1], idx, idx + offu)
    hi0 = lax.bitcast_convert_type(hi0u, jnp.int32)
    hi1 = lax.bitcast_convert_type(hi1u, jnp.int32)
    lo0 = lax.bitcast_convert_type(lo0u, jnp.int32)
    lo1 = lax.bitcast_convert_type(lo1u, jnp.int32)

    nneg16 = jnp.full((16,), n_neg, jnp.int32)
    hup, hvp, hun, hvn = _sc_gather()(h, src, dst, hi0, hi1, lo0, lo1, nneg16)

    wblocks = [_full((128, 64)), _full((1, 64)), _full((64, 1)),
               _full((1, 1)), _full((1, 1))]
    wargs = (Wm1, bm1.reshape(1, 64), Wm2, bm2.reshape(1, 1),
             tau.reshape(1, 1))

    sum_pos = pl.pallas_call(
        _tc_score_pos_kernel,
        grid=(N_EDGES // NB,),
        in_specs=[
            pl.BlockSpec((NB, 32), lambda i: (i, 0)),
            pl.BlockSpec((NB, 32), lambda i: (i, 0)),
            pl.BlockSpec((NB, 1), lambda i: (i, 0)),
            pl.BlockSpec((NB, 1), lambda i: (i, 0)),
        ] + wblocks,
        out_specs=pl.BlockSpec((1, 1), lambda i: (0, 0)),
        out_shape=jax.ShapeDtypeStruct((1, 1), f32),
    )(hup, hvp, src.reshape(N_EDGES, 1), dst.reshape(N_EDGES, 1), *wargs)

    sum_neg = pl.pallas_call(
        _tc_score_neg_kernel,
        grid=(M_NEG // NB,),
        in_specs=[
            pl.BlockSpec((NB, 32), lambda i: (i, 0)),
            pl.BlockSpec((NB, 32), lambda i: (i, 0)),
            _full((1, 1)),
        ] + wblocks,
        out_specs=pl.BlockSpec((1, 1), lambda i: (0, 0)),
        out_shape=jax.ShapeDtypeStruct((1, 1), f32),
    )(hun, hvn, n_neg.reshape(1, 1), *wargs)

    n_pos_f = n_pos.astype(f32)
    n_neg_f = n_neg.astype(f32)
    pos_weight = n_neg_f / n_pos_f
    total = -pos_weight * sum_pos[0, 0] - sum_neg[0, 0]
    recon = total / (n_pos_f + n_neg_f)
    klv = kl[0, 0]
    loss = recon + BETA * klv
    return (loss, recon, klv)


def _threefry2x32(k0, k1, x0, x1):
    rots = ((13, 15, 26, 6), (17, 29, 16, 24))

    def rotl(v, d):
        return (v << jnp.uint32(d)) | (v >> jnp.uint32(32 - d))

    ks = (k0, k1, k0 ^ k1 ^ jnp.uint32(0x1BD11BDA))
    x0 = x0 + ks[0]
    x1 = x1 + ks[1]
    for i in range(5):
        for r in rots[i % 2]:
            x0 = x0 + x1
            x1 = rotl(x1, r)
            x1 = x1 ^ x0
        x0 = x0 + ks[(i + 1) % 3]
        x1 = x1 + ks[(i + 2) % 3] + jnp.uint32(i + 1)
    return x0, x1
